# baseline (speedup 1.0000x reference)
import functools

import jax
import jax.numpy as jnp
from jax.experimental import pallas as pl
from jax.experimental.pallas import tpu as pltpu
from jax.experimental.pallas import tpu_sc as plsc

K = 8
ALPHA = 0.01
CAPACITY_FACTOR = 1.25
B = 1024
NW = 32


def _make_dense(n_experts, grid):
    def _body(x_ref, w_ref, b_ref, gate_ref, psum_ref, acc_p):
        step = pl.program_id(0)

        @pl.when(step == 0)
        def _init():
            acc_p[...] = jnp.zeros_like(acc_p)

        logits = jnp.dot(x_ref[...], w_ref[...],
                         preferred_element_type=jnp.float32)
        lt = jnp.transpose(logits) + b_ref[...]
        m = jnp.max(lt, axis=0, keepdims=True)
        e = jnp.exp(lt - m)
        gate = e / jnp.sum(e, axis=0, keepdims=True)
        acc_p[...] += jnp.sum(gate, axis=1, keepdims=True)
        gate_ref[...] = gate
        psum_ref[...] = acc_p[...]

    return _body


def _dense(x, W, b):
    T, D = x.shape
    N = W.shape[1]
    grid = T // B
    return pl.pallas_call(
        _make_dense(N, grid),
        grid=(grid,),
        in_specs=[
            pl.BlockSpec((B, D), lambda i: (i, 0)),
            pl.BlockSpec((D, N), lambda i: (0, 0)),
            pl.BlockSpec((N, 1), lambda i: (0, 0)),
        ],
        out_specs=[
            pl.BlockSpec((N, B), lambda i: (0, i)),
            pl.BlockSpec((N, 1), lambda i: (0, 0)),
        ],
        out_shape=[
            jax.ShapeDtypeStruct((N, T), jnp.float32),
            jax.ShapeDtypeStruct((N, 1), jnp.float32),
        ],
        scratch_shapes=[pltpu.VMEM((N, 1), jnp.float32)],
        compiler_params=pltpu.CompilerParams(
            dimension_semantics=("arbitrary",)),
    )(x, W, b.reshape(N, 1))


def _make_peel(T, n_experts):
    TPW = T // NW
    H = TPW // 2
    N = n_experts
    mesh = plsc.VectorSubcoreMesh(core_axis_name="c", subcore_axis_name="s")

    @functools.partial(
        pl.kernel, mesh=mesh,
        out_type=[
            jax.ShapeDtypeStruct((K, T), jnp.float32),
            jax.ShapeDtypeStruct((K, T), jnp.int32),
        ],
        scratch_types=[
            pltpu.VMEM((N, H), jnp.float32),
            pltpu.VMEM((K, H), jnp.float32),
            pltpu.VMEM((K, H), jnp.int32),
        ],
    )
    def peel(gate_hbm, probs_hbm, idx_hbm, g_v, op_v, oi_v):
        cid = jax.lax.axis_index("c")
        sid = jax.lax.axis_index("s")
        base = (sid * 2 + cid) * TPW

        for h in range(2):
            off = base + h * H
            pltpu.sync_copy(gate_hbm.at[:, pl.ds(off, H)], g_v)

            def group(jj, _):
                neg = jnp.full((16,), -1.0, jnp.float32)
                zi = jnp.zeros((16,), jnp.int32)
                carry0 = ((neg,) * K + (zi,) * K) * 2

                def ins(e, carry):
                    out = []
                    for g in range(2):
                        bv = list(carry[2 * K * g:2 * K * g + K])
                        bi = list(carry[2 * K * g + K:2 * K * g + 2 * K])
                        v = g_v[e, pl.ds((jj * 2 + g) * 16, 16)]
                        iv = jnp.full((16,), e, jnp.int32)
                        for r in range(K):
                            gt = v > bv[r]
                            bv[r], v = (jnp.where(gt, v, bv[r]),
                                        jnp.where(gt, bv[r], v))
                            bi[r], iv = (jnp.where(gt, iv, bi[r]),
                                         jnp.where(gt, bi[r], iv))
                        out.extend(bv + bi)
                    return tuple(out)

                res = jax.lax.fori_loop(0, N, ins, carry0)
                for g in range(2):
                    for r in range(K):
                        op_v[r, pl.ds((jj * 2 + g) * 16, 16)] = \
                            res[2 * K * g + r]
                        oi_v[r, pl.ds((jj * 2 + g) * 16, 16)] = \
                            res[2 * K * g + K + r]
                return 0
            jax.lax.fori_loop(0, H // 32, group, 0)

            pltpu.sync_copy(op_v, probs_hbm.at[:, pl.ds(off, H)])
            pltpu.sync_copy(oi_v, idx_hbm.at[:, pl.ds(off, H)])

    return peel


def _make_maskloss(n_experts, n_tokens, cap, grid):
    def _body(idx_ref, psum_ref, psum2_ref, mask_ref, loss_ref, acc_a,
              tri_ref):
        step = pl.program_id(0)
        N = n_experts

        @pl.when(step == 0)
        def _init():
            acc_a[...] = jnp.zeros_like(acc_a)
            row = jax.lax.broadcasted_iota(jnp.int32, (B, B), 0)
            col = jax.lax.broadcasted_iota(jnp.int32, (B, B), 1)
            tri_ref[...] = (row <= col).astype(jnp.bfloat16)

        es = jax.lax.broadcasted_iota(jnp.int32, (N, B), 0)
        act = jnp.zeros((N, B), jnp.float32)
        for r in range(K):
            act = act + jnp.where((es - idx_ref[r:r + 1, :]) == 0, 1.0, 0.0)
        carry = acc_a[...]
        csum = jnp.dot(act.astype(jnp.bfloat16), tri_ref[...],
                       preferred_element_type=jnp.float32)
        acc_a[...] = carry + csum[:, B - 1:B]
        rank = csum + carry
        mask_ref[...] = (act > 0.5) & (rank <= cap)

        @pl.when(step == grid - 1)
        def _fin():
            s = jnp.sum((psum_ref[...] + psum2_ref[...]) * acc_a[...])
            loss_ref[0, 0] = ALPHA * N * s / (n_tokens * n_tokens)

    return _body


def _maskloss(idx_t, psum, psum2, n_experts):
    N = n_experts
    T = idx_t.shape[1]
    cap = int(CAPACITY_FACTOR * T / N)
    grid = T // B
    return pl.pallas_call(
        _make_maskloss(N, T, cap, grid),
        grid=(grid,),
        in_specs=[
            pl.BlockSpec((K, B), lambda i: (0, i)),
            pl.BlockSpec((N, 1), lambda i: (0, 0)),
            pl.BlockSpec((N, 1), lambda i: (0, 0)),
        ],
        out_specs=[
            pl.BlockSpec((N, B), lambda i: (0, i)),
            pl.BlockSpec((1, 1), lambda i: (0, 0),
                         memory_space=pltpu.SMEM),
        ],
        out_shape=[
            jax.ShapeDtypeStruct((N, T), jnp.bool_),
            jax.ShapeDtypeStruct((1, 1), jnp.float32),
        ],
        scratch_shapes=[
            pltpu.VMEM((N, 1), jnp.float32),
            pltpu.VMEM((B, B), jnp.bfloat16),
        ],
        compiler_params=pltpu.CompilerParams(
            dimension_semantics=("arbitrary",)),
    )(idx_t, psum, psum2)


def kernel(x, W, b):
    T = x.shape[0]
    N = W.shape[1]
    Th = T // 2
    peel = _make_peel(Th, N)
    gate0, psum0 = _dense(x[:Th], W, b)
    probs0, idx0 = peel(gate0)
    gate1, psum1 = _dense(x[Th:], W, b)
    probs1, idx1 = peel(gate1)
    probs_t = jnp.concatenate([probs0, probs1], axis=1)
    idx_t = jnp.concatenate([idx0, idx1], axis=1)
    mask, loss = _maskloss(idx_t, psum0, psum1, N)
    return (loss[0, 0], jnp.transpose(probs_t), jnp.transpose(idx_t), mask)


# speedup vs baseline: 2.2588x; 2.2588x over previous
import functools

import jax
import jax.numpy as jnp
from jax.experimental import pallas as pl
from jax.experimental.pallas import tpu as pltpu
from jax.experimental.pallas import tpu_sc as plsc

K = 8
ALPHA = 0.01
CAPACITY_FACTOR = 1.25
B = 1024
NW = 32


def _make_dense(n_experts, grid):
    def _body(x_ref, w_ref, b_ref, gate_ref, psum_ref, acc_p):
        step = pl.program_id(0)

        @pl.when(step == 0)
        def _init():
            acc_p[...] = jnp.zeros_like(acc_p)

        logits = jnp.dot(x_ref[...], w_ref[...],
                         preferred_element_type=jnp.float32)
        lt = jnp.transpose(logits) + b_ref[...]
        m = jnp.max(lt, axis=0, keepdims=True)
        e = jnp.exp(lt - m)
        gate = e / jnp.sum(e, axis=0, keepdims=True)
        acc_p[...] += jnp.sum(gate, axis=1, keepdims=True)
        gate_ref[...] = gate
        psum_ref[...] = acc_p[...]

    return _body


def _dense(x, W, b):
    T, D = x.shape
    N = W.shape[1]
    grid = T // B
    return pl.pallas_call(
        _make_dense(N, grid),
        grid=(grid,),
        in_specs=[
            pl.BlockSpec((B, D), lambda i: (i, 0)),
            pl.BlockSpec((D, N), lambda i: (0, 0)),
            pl.BlockSpec((N, 1), lambda i: (0, 0)),
        ],
        out_specs=[
            pl.BlockSpec((N, B), lambda i: (0, i)),
            pl.BlockSpec((N, 1), lambda i: (0, 0)),
        ],
        out_shape=[
            jax.ShapeDtypeStruct((N, T), jnp.float32),
            jax.ShapeDtypeStruct((N, 1), jnp.float32),
        ],
        scratch_shapes=[pltpu.VMEM((N, 1), jnp.float32)],
        compiler_params=pltpu.CompilerParams(
            dimension_semantics=("arbitrary",)),
    )(x, W, b.reshape(N, 1))


def _make_peel(T, n_experts):
    TPW = T // NW
    H = TPW // 2
    N = n_experts
    mesh = plsc.VectorSubcoreMesh(core_axis_name="c", subcore_axis_name="s")

    @functools.partial(
        pl.kernel, mesh=mesh,
        out_type=[
            jax.ShapeDtypeStruct((K, T), jnp.float32),
            jax.ShapeDtypeStruct((K, T), jnp.int32),
        ],
        scratch_types=[
            pltpu.VMEM((N, H), jnp.float32),
            pltpu.VMEM((K, H), jnp.float32),
            pltpu.VMEM((K, H), jnp.int32),
        ],
    )
    def peel(gate_hbm, probs_hbm, idx_hbm, g_v, op_v, oi_v):
        cid = jax.lax.axis_index("c")
        sid = jax.lax.axis_index("s")
        base = (sid * 2 + cid) * TPW

        for h in range(2):
            off = base + h * H
            pltpu.sync_copy(gate_hbm.at[:, pl.ds(off, H)], g_v)

            def group(jj, _):
                neg = jnp.full((16,), -1.0, jnp.float32)
                zi = jnp.zeros((16,), jnp.int32)
                carry0 = ((neg,) * K + (zi,) * K) * 2

                def ins(e, carry):
                    out = []
                    for g in range(2):
                        bv = list(carry[2 * K * g:2 * K * g + K])
                        bi = list(carry[2 * K * g + K:2 * K * g + 2 * K])
                        v = g_v[e, pl.ds((jj * 2 + g) * 16, 16)]
                        iv = jnp.full((16,), e, jnp.int32)
                        for r in range(K):
                            gt = v > bv[r]
                            bv[r], v = (jnp.where(gt, v, bv[r]),
                                        jnp.where(gt, bv[r], v))
                            bi[r], iv = (jnp.where(gt, iv, bi[r]),
                                         jnp.where(gt, bi[r], iv))
                        out.extend(bv + bi)
                    return tuple(out)

                res = jax.lax.fori_loop(0, N, ins, carry0)
                for g in range(2):
                    for r in range(K):
                        op_v[r, pl.ds((jj * 2 + g) * 16, 16)] = \
                            res[2 * K * g + r]
                        oi_v[r, pl.ds((jj * 2 + g) * 16, 16)] = \
                            res[2 * K * g + K + r]
                return 0
            jax.lax.fori_loop(0, H // 32, group, 0)

            pltpu.sync_copy(op_v, probs_hbm.at[:, pl.ds(off, H)])
            pltpu.sync_copy(oi_v, idx_hbm.at[:, pl.ds(off, H)])

    return peel


def _make_maskloss(n_experts, n_tokens, cap, grid):
    def _body(idx_ref, psum_ref, mask_ref, loss_ref, acc_a, tri_ref):
        step = pl.program_id(0)
        N = n_experts

        @pl.when(step == 0)
        def _init():
            acc_a[...] = jnp.zeros_like(acc_a)
            row = jax.lax.broadcasted_iota(jnp.int32, (B, B), 0)
            col = jax.lax.broadcasted_iota(jnp.int32, (B, B), 1)
            tri_ref[...] = (row <= col).astype(jnp.bfloat16)

        es = jax.lax.broadcasted_iota(jnp.int32, (N, B), 0)
        act = jnp.zeros((N, B), jnp.float32)
        for r in range(K):
            act = act + jnp.where((es - idx_ref[r:r + 1, :]) == 0, 1.0, 0.0)
        carry = acc_a[...]
        csum = jnp.dot(act.astype(jnp.bfloat16), tri_ref[...],
                       preferred_element_type=jnp.float32)
        acc_a[...] = carry + csum[:, B - 1:B]
        rank = csum + carry
        mask_ref[...] = (act > 0.5) & (rank <= cap)

        @pl.when(step == grid - 1)
        def _fin():
            s = jnp.sum(psum_ref[...] * acc_a[...])
            loss_ref[0, 0] = ALPHA * N * s / (n_tokens * n_tokens)

    return _body


def _maskloss(idx_t, psum, n_experts):
    N = n_experts
    T = idx_t.shape[1]
    cap = int(CAPACITY_FACTOR * T / N)
    grid = T // B
    return pl.pallas_call(
        _make_maskloss(N, T, cap, grid),
        grid=(grid,),
        in_specs=[
            pl.BlockSpec((K, B), lambda i: (0, i)),
            pl.BlockSpec((N, 1), lambda i: (0, 0)),
        ],
        out_specs=[
            pl.BlockSpec((N, B), lambda i: (0, i)),
            pl.BlockSpec((1, 1), lambda i: (0, 0),
                         memory_space=pltpu.SMEM),
        ],
        out_shape=[
            jax.ShapeDtypeStruct((N, T), jnp.bool_),
            jax.ShapeDtypeStruct((1, 1), jnp.float32),
        ],
        scratch_shapes=[
            pltpu.VMEM((N, 1), jnp.float32),
            pltpu.VMEM((B, B), jnp.bfloat16),
        ],
        compiler_params=pltpu.CompilerParams(
            dimension_semantics=("arbitrary",)),
    )(idx_t, psum)


def kernel(x, W, b):
    T = x.shape[0]
    N = W.shape[1]
    gate_t, psum = _dense(x, W, b)
    probs_t, idx_t = _make_peel(T, N)(gate_t)
    mask, loss = _maskloss(idx_t, psum, N)
    return (loss[0, 0], jnp.transpose(probs_t), jnp.transpose(idx_t), mask)
